# baseline (speedup 1.0000x reference)
import functools

import jax
import jax.numpy as jnp
from jax import lax
from jax.experimental import pallas as pl
from jax.experimental.pallas import tpu as pltpu
from jax.experimental.pallas import tpu_sc as plsc

VOCAB = 100000
EMB = 128
CTX = 200
HIDDEN = 200

BLK = 8192
NB = (VOCAB + BLK - 1) // BLK

B_PAD = 256


def _sc_gather(emb_table, idx_pad):
    info = plsc.get_sparse_core_info()
    ncores = info.num_cores
    nw = ncores * info.num_subcores
    b_per_w = B_PAD // nw

    mesh = plsc.VectorSubcoreMesh(core_axis_name="c", subcore_axis_name="s")

    @functools.partial(
        pl.kernel,
        out_type=jax.ShapeDtypeStruct((B_PAD, EMB), jnp.float32),
        mesh=mesh,
        scratch_types=[
            pltpu.VMEM((b_per_w,), jnp.int32),
            pltpu.VMEM((b_per_w, EMB), jnp.float32),
            pltpu.SemaphoreType.DMA,
        ],
    )
    def gather_kernel(table_hbm, idx_hbm, out_hbm, idx_v, rows_v, sem):
        wid = lax.axis_index("s") * ncores + lax.axis_index("c")
        base = wid * b_per_w
        pltpu.sync_copy(idx_hbm.at[pl.ds(base, b_per_w)], idx_v)
        pltpu.async_copy(table_hbm.at[idx_v], rows_v, sem).wait()
        pltpu.sync_copy(rows_v, out_hbm.at[pl.ds(base, b_per_w)])

    return gather_kernel(emb_table, idx_pad)


def _mlp_body(emb_ref, fcw_ref, fcb_ref, outwt_ref, outb_ref, out_ref,
              logits_ref, x_ref, lse_ref):
    i = pl.program_id(0)

    @pl.when(i == 0)
    def _fc():
        x = lax.dot_general(
            emb_ref[...], fcw_ref[...],
            dimension_numbers=(((1,), (1,)), ((), ())),
            preferred_element_type=jnp.float32)
        x_ref[...] = jnp.maximum(x + fcb_ref[...], 0.0)

    @pl.when((i >= 1) & (i < 1 + NB))
    def _proj():
        j = i - 1
        lg = lax.dot_general(
            x_ref[...], outwt_ref[...],
            dimension_numbers=(((1,), (0,)), ((), ())),
            preferred_element_type=jnp.float32)
        logits_ref[j] = lg + outb_ref[...]

    @pl.when(i == 1 + NB)
    def _lse():
        lg = logits_ref[...]
        blk_id = lax.broadcasted_iota(jnp.int32, (NB, 1, BLK), 0)
        lane = lax.broadcasted_iota(jnp.int32, (NB, 1, BLK), 2)
        valid = blk_id * BLK + lane < VOCAB
        lg = jnp.where(valid, lg, -1e30)
        m = jnp.max(lg)
        s = jnp.sum(jnp.exp(lg - m))
        lse_ref[...] = jnp.broadcast_to(m + jnp.log(s), (1, BLK))

    @pl.when(i > 1 + NB)
    def _write():
        j2 = i - (NB + 2)
        out_ref[...] = logits_ref[j2] - lse_ref[...]


def _mlp(embeds, fc_w, fc_b, out_wt, out_b):
    grid = (2 * NB + 2,)
    return pl.pallas_call(
        _mlp_body,
        grid=grid,
        in_specs=[
            pl.BlockSpec(memory_space=pltpu.MemorySpace.VMEM),
            pl.BlockSpec(memory_space=pltpu.MemorySpace.VMEM),
            pl.BlockSpec(memory_space=pltpu.MemorySpace.VMEM),
            pl.BlockSpec((HIDDEN, BLK), lambda i: (0, jnp.clip(i - 1, 0, NB - 1))),
            pl.BlockSpec((1, BLK), lambda i: (0, jnp.clip(i - 1, 0, NB - 1))),
        ],
        out_specs=pl.BlockSpec(
            (1, BLK), lambda i: (0, jnp.clip(i - (NB + 2), 0, NB - 1))),
        out_shape=jax.ShapeDtypeStruct((1, VOCAB), jnp.float32),
        scratch_shapes=[
            pltpu.VMEM((NB, 1, BLK), jnp.float32),
            pltpu.VMEM((1, HIDDEN), jnp.float32),
            pltpu.VMEM((1, BLK), jnp.float32),
        ],
    )(embeds, fc_w, fc_b, out_wt, out_b)


def kernel(input, emb_table, fc_w, fc_b, out_w, out_b):
    idx_pad = jnp.pad(input.astype(jnp.int32), (0, B_PAD - CTX))
    rows = _sc_gather(emb_table, idx_pad)
    embeds = rows[:CTX].reshape(1, CTX * EMB)
    return _mlp(
        embeds,
        fc_w,
        fc_b.reshape(1, HIDDEN),
        out_w.T,
        out_b.reshape(1, VOCAB),
    )


# speedup vs baseline: 1.0078x; 1.0078x over previous
import functools

import jax
import jax.numpy as jnp
from jax import lax
from jax.experimental import pallas as pl
from jax.experimental.pallas import tpu as pltpu
from jax.experimental.pallas import tpu_sc as plsc

VOCAB = 100000
EMB = 128
CTX = 200
HIDDEN = 200

BLK = 8192
NB = (VOCAB + BLK - 1) // BLK
NB2 = (NB + 1) // 2

B_PAD = 256


def _sc_gather(emb_table, idx_pad):
    info = plsc.get_sparse_core_info()
    ncores = info.num_cores
    nw = ncores * info.num_subcores
    b_per_w = B_PAD // nw

    mesh = plsc.VectorSubcoreMesh(core_axis_name="c", subcore_axis_name="s")

    @functools.partial(
        pl.kernel,
        out_type=jax.ShapeDtypeStruct((B_PAD, EMB), jnp.float32),
        mesh=mesh,
        scratch_types=[
            pltpu.VMEM((b_per_w,), jnp.int32),
            pltpu.VMEM((b_per_w, EMB), jnp.float32),
            pltpu.SemaphoreType.DMA,
        ],
    )
    def gather_kernel(table_hbm, idx_hbm, out_hbm, idx_v, rows_v, sem):
        wid = lax.axis_index("s") * ncores + lax.axis_index("c")
        base = wid * b_per_w
        pltpu.sync_copy(idx_hbm.at[pl.ds(base, b_per_w)], idx_v)
        pltpu.async_copy(table_hbm.at[idx_v], rows_v, sem).wait()
        pltpu.sync_copy(rows_v, out_hbm.at[pl.ds(base, b_per_w)])

    return gather_kernel(emb_table, idx_pad)


def _mlp_body(emb_ref, fcw_ref, fcb_ref, outwt_a_ref, outb_a_ref,
              outwt_b_ref, outb_b_ref, out_ref,
              logits_ref, x_ref, lse_ref):
    i = pl.program_id(0)

    @pl.when(i == 0)
    def _fc():
        x = lax.dot_general(
            emb_ref[...], fcw_ref[...],
            dimension_numbers=(((1,), (1,)), ((), ())),
            preferred_element_type=jnp.float32)
        x_ref[...] = jnp.maximum(x + fcb_ref[...], 0.0)

    @pl.when((i >= 1) & (i < 1 + NB2))
    def _proj():
        j = i - 1
        lga = lax.dot_general(
            x_ref[...], outwt_a_ref[...],
            dimension_numbers=(((1,), (0,)), ((), ())),
            preferred_element_type=jnp.float32)
        logits_ref[2 * j] = lga + outb_a_ref[...]

        @pl.when(2 * j + 1 < NB)
        def _():
            lgb = lax.dot_general(
                x_ref[...], outwt_b_ref[...],
                dimension_numbers=(((1,), (0,)), ((), ())),
                preferred_element_type=jnp.float32)
            logits_ref[2 * j + 1] = lgb + outb_b_ref[...]

    @pl.when(i == 1 + NB2)
    def _lse():
        lg = logits_ref[...]
        blk_id = lax.broadcasted_iota(jnp.int32, (NB, 1, BLK), 0)
        lane = lax.broadcasted_iota(jnp.int32, (NB, 1, BLK), 2)
        valid = blk_id * BLK + lane < VOCAB
        lg = jnp.where(valid, lg, -1e30)
        m = jnp.max(lg)
        s = jnp.sum(jnp.exp(lg - m))
        lse_ref[...] = jnp.broadcast_to(m + jnp.log(s), (1, BLK))

    @pl.when(i > 1 + NB2)
    def _write():
        j2 = i - (NB2 + 2)
        out_ref[...] = logits_ref[j2] - lse_ref[...]


def _mlp(embeds, fc_w, fc_b, out_wt, out_b):
    grid = (NB2 + 2 + NB,)
    even = lambda i: (0, jnp.clip(2 * (i - 1), 0, NB - 1))
    odd = lambda i: (0, jnp.clip(2 * (i - 1) + 1, 0, NB - 1))
    return pl.pallas_call(
        _mlp_body,
        grid=grid,
        in_specs=[
            pl.BlockSpec(memory_space=pltpu.MemorySpace.VMEM),
            pl.BlockSpec(memory_space=pltpu.MemorySpace.VMEM),
            pl.BlockSpec(memory_space=pltpu.MemorySpace.VMEM),
            pl.BlockSpec((HIDDEN, BLK), even),
            pl.BlockSpec((1, BLK), even),
            pl.BlockSpec((HIDDEN, BLK), odd),
            pl.BlockSpec((1, BLK), odd),
        ],
        out_specs=pl.BlockSpec(
            (1, BLK), lambda i: (0, jnp.clip(i - (NB2 + 2), 0, NB - 1))),
        out_shape=jax.ShapeDtypeStruct((1, VOCAB), jnp.float32),
        scratch_shapes=[
            pltpu.VMEM((NB, 1, BLK), jnp.float32),
            pltpu.VMEM((1, HIDDEN), jnp.float32),
            pltpu.VMEM((1, BLK), jnp.float32),
        ],
    )(embeds, fc_w, fc_b, out_wt, out_b, out_wt, out_b)


def kernel(input, emb_table, fc_w, fc_b, out_w, out_b):
    idx_pad = jnp.pad(input.astype(jnp.int32), (0, B_PAD - CTX))
    rows = _sc_gather(emb_table, idx_pad)
    embeds = rows[:CTX].reshape(1, CTX * EMB)
    return _mlp(
        embeds,
        fc_w,
        fc_b.reshape(1, HIDDEN),
        out_w.T,
        out_b.reshape(1, VOCAB),
    )


# speedup vs baseline: 1.0125x; 1.0047x over previous
import functools

import jax
import jax.numpy as jnp
from jax import lax
from jax.experimental import pallas as pl
from jax.experimental.pallas import tpu as pltpu
from jax.experimental.pallas import tpu_sc as plsc

VOCAB = 100000
EMB = 128
CTX = 200
HIDDEN = 200

BLK = 16384
NB = (VOCAB + BLK - 1) // BLK

B_PAD = 256


def _sc_gather(emb_table, idx_pad):
    info = plsc.get_sparse_core_info()
    ncores = info.num_cores
    nw = ncores * info.num_subcores
    b_per_w = B_PAD // nw

    mesh = plsc.VectorSubcoreMesh(core_axis_name="c", subcore_axis_name="s")

    @functools.partial(
        pl.kernel,
        out_type=jax.ShapeDtypeStruct((B_PAD, EMB), jnp.float32),
        mesh=mesh,
        scratch_types=[
            pltpu.VMEM((b_per_w,), jnp.int32),
            pltpu.VMEM((b_per_w, EMB), jnp.float32),
            pltpu.SemaphoreType.DMA,
        ],
    )
    def gather_kernel(table_hbm, idx_hbm, out_hbm, idx_v, rows_v, sem):
        wid = lax.axis_index("s") * ncores + lax.axis_index("c")
        base = wid * b_per_w
        pltpu.sync_copy(idx_hbm.at[pl.ds(base, b_per_w)], idx_v)
        pltpu.async_copy(table_hbm.at[idx_v], rows_v, sem).wait()
        pltpu.sync_copy(rows_v, out_hbm.at[pl.ds(base, b_per_w)])

    return gather_kernel(emb_table, idx_pad)


def _mlp_body(emb_ref, fcw_ref, fcb_ref, outwt_ref, outb_ref, out_ref,
              logits_ref, x_ref, lse_ref):
    i = pl.program_id(0)

    @pl.when(i == 0)
    def _fc():
        x = lax.dot_general(
            emb_ref[...], fcw_ref[...],
            dimension_numbers=(((1,), (1,)), ((), ())),
            preferred_element_type=jnp.float32)
        x_ref[...] = jnp.maximum(x + fcb_ref[...], 0.0)

    @pl.when((i >= 1) & (i < 1 + NB))
    def _proj():
        j = i - 1
        lg = lax.dot_general(
            x_ref[...], outwt_ref[...],
            dimension_numbers=(((1,), (0,)), ((), ())),
            preferred_element_type=jnp.float32)
        logits_ref[j] = lg + outb_ref[...]

    @pl.when(i == 1 + NB)
    def _lse():
        lg = logits_ref[...]
        blk_id = lax.broadcasted_iota(jnp.int32, (NB, 1, BLK), 0)
        lane = lax.broadcasted_iota(jnp.int32, (NB, 1, BLK), 2)
        valid = blk_id * BLK + lane < VOCAB
        lg = jnp.where(valid, lg, -1e30)
        m = jnp.max(lg)
        s = jnp.sum(jnp.exp(lg - m))
        lse_ref[...] = jnp.broadcast_to(m + jnp.log(s), (1, BLK))

    @pl.when(i > 1 + NB)
    def _write():
        j2 = i - (NB + 2)
        out_ref[...] = logits_ref[j2] - lse_ref[...]


def _mlp(embeds, fc_w, fc_b, out_wt, out_b):
    grid = (2 * NB + 2,)
    return pl.pallas_call(
        _mlp_body,
        grid=grid,
        in_specs=[
            pl.BlockSpec(memory_space=pltpu.MemorySpace.VMEM),
            pl.BlockSpec(memory_space=pltpu.MemorySpace.VMEM),
            pl.BlockSpec(memory_space=pltpu.MemorySpace.VMEM),
            pl.BlockSpec((HIDDEN, BLK), lambda i: (0, jnp.clip(i - 1, 0, NB - 1))),
            pl.BlockSpec((1, BLK), lambda i: (0, jnp.clip(i - 1, 0, NB - 1))),
        ],
        out_specs=pl.BlockSpec(
            (1, BLK), lambda i: (0, jnp.clip(i - (NB + 2), 0, NB - 1))),
        out_shape=jax.ShapeDtypeStruct((1, VOCAB), jnp.float32),
        scratch_shapes=[
            pltpu.VMEM((NB, 1, BLK), jnp.float32),
            pltpu.VMEM((1, HIDDEN), jnp.float32),
            pltpu.VMEM((1, BLK), jnp.float32),
        ],
    )(embeds, fc_w, fc_b, out_wt, out_b)


def kernel(input, emb_table, fc_w, fc_b, out_w, out_b):
    idx_pad = jnp.pad(input.astype(jnp.int32), (0, B_PAD - CTX))
    rows = _sc_gather(emb_table, idx_pad)
    embeds = rows[:CTX].reshape(1, CTX * EMB)
    return _mlp(
        embeds,
        fc_w,
        fc_b.reshape(1, HIDDEN),
        out_w.T,
        out_b.reshape(1, VOCAB),
    )


# speedup vs baseline: 1.0915x; 1.0780x over previous
import functools

import jax
import jax.numpy as jnp
from jax import lax
from jax.experimental import pallas as pl
from jax.experimental.pallas import tpu as pltpu
from jax.experimental.pallas import tpu_sc as plsc

VOCAB = 100000
EMB = 128
CTX = 200
HIDDEN = 200

BLK = 16384
NB = (VOCAB + BLK - 1) // BLK

B_PER_W = 8
NW_USED = CTX // B_PER_W


def _sc_gather(emb_table, idx):
    info = plsc.get_sparse_core_info()
    ncores = info.num_cores

    mesh = plsc.VectorSubcoreMesh(core_axis_name="c", subcore_axis_name="s")

    @functools.partial(
        pl.kernel,
        out_type=jax.ShapeDtypeStruct((CTX, EMB), jnp.float32),
        mesh=mesh,
        scratch_types=[
            pltpu.VMEM((B_PER_W,), jnp.int32),
            pltpu.VMEM((B_PER_W, EMB), jnp.float32),
            pltpu.SemaphoreType.DMA,
        ],
    )
    def gather_kernel(table_hbm, idx_hbm, out_hbm, idx_v, rows_v, sem):
        wid = lax.axis_index("s") * ncores + lax.axis_index("c")

        @pl.when(wid < NW_USED)
        def _():
            base = wid * B_PER_W
            pltpu.sync_copy(idx_hbm.at[pl.ds(base, B_PER_W)], idx_v)
            pltpu.async_copy(table_hbm.at[idx_v], rows_v, sem).wait()
            pltpu.sync_copy(rows_v, out_hbm.at[pl.ds(base, B_PER_W)])

    return gather_kernel(emb_table, idx)


def _mlp_body(emb_ref, fcw_ref, fcb_ref, outwt_ref, outb_ref, out_ref,
              logits_ref, x_ref, lse_ref):
    i = pl.program_id(0)

    @pl.when(i == 0)
    def _fc():
        x = lax.dot_general(
            emb_ref[...], fcw_ref[...],
            dimension_numbers=(((1,), (1,)), ((), ())),
            preferred_element_type=jnp.float32)
        x_ref[...] = jnp.maximum(x + fcb_ref[...], 0.0)

    @pl.when((i >= 1) & (i < 1 + NB))
    def _proj():
        j = i - 1
        lg = lax.dot_general(
            x_ref[...], outwt_ref[...],
            dimension_numbers=(((1,), (0,)), ((), ())),
            preferred_element_type=jnp.float32)
        logits_ref[j] = lg + outb_ref[...].reshape(1, BLK)

    @pl.when(i == 1 + NB)
    def _lse():
        lg = logits_ref[...]
        blk_id = lax.broadcasted_iota(jnp.int32, (NB, 1, BLK), 0)
        lane = lax.broadcasted_iota(jnp.int32, (NB, 1, BLK), 2)
        valid = blk_id * BLK + lane < VOCAB
        lg = jnp.where(valid, lg, -1e30)
        m = jnp.max(lg)
        s = jnp.sum(jnp.exp(lg - m))
        lse_ref[...] = jnp.broadcast_to(m + jnp.log(s), (1, BLK))

    @pl.when(i > 1 + NB)
    def _write():
        j2 = i - (NB + 2)
        out_ref[...] = logits_ref[j2] - lse_ref[...]


def _mlp(embeds, fc_w, fc_b, out_wt, out_b):
    grid = (2 * NB + 2,)
    return pl.pallas_call(
        _mlp_body,
        grid=grid,
        in_specs=[
            pl.BlockSpec(memory_space=pltpu.MemorySpace.VMEM),
            pl.BlockSpec(memory_space=pltpu.MemorySpace.VMEM),
            pl.BlockSpec(memory_space=pltpu.MemorySpace.VMEM),
            pl.BlockSpec((HIDDEN, BLK), lambda i: (0, jnp.clip(i - 1, 0, NB - 1))),
            pl.BlockSpec((BLK,), lambda i: (jnp.clip(i - 1, 0, NB - 1),)),
        ],
        out_specs=pl.BlockSpec(
            (1, BLK), lambda i: (0, jnp.clip(i - (NB + 2), 0, NB - 1))),
        out_shape=jax.ShapeDtypeStruct((1, VOCAB), jnp.float32),
        scratch_shapes=[
            pltpu.VMEM((NB, 1, BLK), jnp.float32),
            pltpu.VMEM((1, HIDDEN), jnp.float32),
            pltpu.VMEM((1, BLK), jnp.float32),
        ],
    )(embeds, fc_w, fc_b, out_wt, out_b)


def kernel(input, emb_table, fc_w, fc_b, out_w, out_b):
    rows = _sc_gather(emb_table, input.astype(jnp.int32))
    embeds = rows.reshape(1, CTX * EMB)
    return _mlp(
        embeds,
        fc_w,
        fc_b.reshape(1, HIDDEN),
        out_w.T,
        out_b,
    )


# speedup vs baseline: 1.1192x; 1.0254x over previous
import functools

import jax
import jax.numpy as jnp
from jax import lax
from jax.experimental import pallas as pl
from jax.experimental.pallas import tpu as pltpu
from jax.experimental.pallas import tpu_sc as plsc

VOCAB = 100000
EMB = 128
CTX = 200
HIDDEN = 200

ROWS = 40
NCH = HIDDEN // ROWS

B_PER_W = 8
NW_USED = CTX // B_PER_W


def _sc_gather(emb_table, idx):
    info = plsc.get_sparse_core_info()
    ncores = info.num_cores

    mesh = plsc.VectorSubcoreMesh(core_axis_name="c", subcore_axis_name="s")

    @functools.partial(
        pl.kernel,
        out_type=jax.ShapeDtypeStruct((CTX, EMB), jnp.float32),
        mesh=mesh,
        scratch_types=[
            pltpu.VMEM((B_PER_W,), jnp.int32),
            pltpu.VMEM((B_PER_W, EMB), jnp.float32),
            pltpu.SemaphoreType.DMA,
        ],
    )
    def gather_kernel(table_hbm, idx_hbm, out_hbm, idx_v, rows_v, sem):
        wid = lax.axis_index("s") * ncores + lax.axis_index("c")

        @pl.when(wid < NW_USED)
        def _():
            base = wid * B_PER_W
            pltpu.sync_copy(idx_hbm.at[pl.ds(base, B_PER_W)], idx_v)
            pltpu.async_copy(table_hbm.at[idx_v], rows_v, sem).wait()
            pltpu.sync_copy(rows_v, out_hbm.at[pl.ds(base, B_PER_W)])

    return gather_kernel(emb_table, idx)


def _mlp_body(emb_ref, fcw_ref, fcb_ref, outwt_ref, outb_ref, out_ref,
              logits_ref, xt_ref):
    i = pl.program_id(0)

    @pl.when(i == 0)
    def _fc():
        x = lax.dot_general(
            emb_ref[...], fcw_ref[...],
            dimension_numbers=(((1,), (1,)), ((), ())),
            preferred_element_type=jnp.float32)
        xt_ref[...] = jnp.maximum(x + fcb_ref[...], 0.0)

    for jj in range(NCH):
        @pl.when(i == 1 + jj)
        def _proj(jj=jj):
            part = lax.dot_general(
                xt_ref[:, jj * ROWS:(jj + 1) * ROWS], outwt_ref[...],
                dimension_numbers=(((1,), (0,)), ((), ())),
                preferred_element_type=jnp.float32)
            if jj == 0:
                logits_ref[...] = part + outb_ref[...].reshape(1, VOCAB)
            else:
                logits_ref[...] = logits_ref[...] + part

    @pl.when(i == 1 + NCH)
    def _finish():
        lg = logits_ref[...]
        m = jnp.max(lg)
        s = jnp.sum(jnp.exp(lg - m))
        out_ref[...] = lg - (m + jnp.log(s))


def _mlp(embeds, fc_w, fc_b, out_wt, out_b):
    grid = (NCH + 2,)
    return pl.pallas_call(
        _mlp_body,
        grid=grid,
        in_specs=[
            pl.BlockSpec(memory_space=pltpu.MemorySpace.VMEM),
            pl.BlockSpec(memory_space=pltpu.MemorySpace.VMEM),
            pl.BlockSpec(memory_space=pltpu.MemorySpace.VMEM),
            pl.BlockSpec((ROWS, VOCAB), lambda i: (jnp.clip(i - 1, 0, NCH - 1), 0)),
            pl.BlockSpec(memory_space=pltpu.MemorySpace.VMEM),
        ],
        out_specs=pl.BlockSpec((1, VOCAB), lambda i: (0, 0)),
        out_shape=jax.ShapeDtypeStruct((1, VOCAB), jnp.float32),
        scratch_shapes=[
            pltpu.VMEM((1, VOCAB), jnp.float32),
            pltpu.VMEM((1, HIDDEN), jnp.float32),
        ],
    )(embeds, fc_w, fc_b, out_wt, out_b)


def kernel(input, emb_table, fc_w, fc_b, out_w, out_b):
    rows = _sc_gather(emb_table, input.astype(jnp.int32))
    embeds = rows.reshape(1, CTX * EMB)
    return _mlp(
        embeds,
        fc_w,
        fc_b.reshape(1, HIDDEN),
        out_w.T,
        out_b,
    )
